# baseline (speedup 1.0000x reference)
import functools

import jax
import jax.numpy as jnp
import numpy as np
from jax import lax
from jax.experimental import pallas as pl
from jax.experimental.pallas import tpu as pltpu
from jax.experimental.pallas import tpu_sc as plsc

QUANT = 20
B = 8
N = 2_000_000
NUM_WORKERS = 32
LANE = 16

COL_TILES = N // 128
TILES_PER_W = COL_TILES // NUM_WORKERS
CT = 8
CHUNK_COLS = CT * 128
NCHUNK = TILES_PER_W // CT
NPAIR = NCHUNK // 2
HIST = B * QUANT * LANE

_INV_STEP = np.float32(1.0) / np.float32(1.0 / QUANT)


def _sc_hist_kernel(x_hbm, out_hbm, buf0, buf1, tail, hist, counts, sem0, sem1):
    wid = lax.axis_index("s") * 2 + lax.axis_index("c")
    wbase = wid * (TILES_PER_W * 128)

    lanes = lax.iota(jnp.int32, 16)
    ones = jnp.ones((16,), jnp.float32)
    all_roffs = [lanes + _r * QUANT * 16 for _r in range(B)]

    zeros = jnp.zeros((16,), jnp.float32)

    @plsc.parallel_loop(0, HIST, step=16, unroll=4)
    def _(i):
        hist[pl.ds(i, 16)] = zeros

    def process(buf, ncols=CHUNK_COLS):
        for r0 in range(0, B, 4):
            @plsc.parallel_loop(0, ncols, step=16, unroll=8)
            def _(i):
                for k in range(r0, r0 + 4):
                    v = buf[k, pl.ds(i, 16)]
                    bins = (v * _INV_STEP).astype(jnp.int32)
                    plsc.addupdate_scatter(hist, [bins * 16 + all_roffs[k]], ones)

    def start(col, buf, sem):
        col = pl.multiple_of(col, 128)
        return pltpu.async_copy(x_hbm.at[:, pl.ds(col, CHUNK_COLS)], buf, sem)

    def drain(buf, sem):
        pltpu.make_async_copy(
            x_hbm.at[:, pl.ds(0, CHUNK_COLS)], buf, sem).wait()

    start(wbase, buf0, sem0)

    def pair_body(j, _):
        col0 = wbase + j * (2 * CHUNK_COLS)
        h1 = start(col0 + CHUNK_COLS, buf1, sem1)
        drain(buf0, sem0)
        process(buf0)
        start(col0 + 2 * CHUNK_COLS, buf0, sem0)
        h1.wait()
        process(buf1)
        return 0

    lax.fori_loop(0, NPAIR, pair_body, 0)
    drain(buf0, sem0)
    process(buf0)

    @pl.when(wid < COL_TILES - NUM_WORKERS * TILES_PER_W)
    def _():
        col = pl.multiple_of((NUM_WORKERS * TILES_PER_W + wid) * 128, 128)
        pltpu.sync_copy(x_hbm.at[:, pl.ds(col, 128)], tail)
        process(tail, 128)

    perms = [lanes * 16 + ((lanes + s) & 15) for s in range(16)]
    for jb in range(B * QUANT // 16):
        base = jb * 256
        acc = plsc.load_gather(hist, [perms[0] + base])
        for s in range(1, 16):
            acc = acc + plsc.load_gather(hist, [perms[s] + base])
        counts[pl.ds(jb * 16, 16)] = acc

    pltpu.sync_copy(counts, out_hbm.at[pl.ds(wid * (B * QUANT), B * QUANT)])


def _entropy_kernel(p_ref, o_ref):
    counts = jnp.sum(p_ref[...], axis=0, keepdims=True)
    prob = counts * jnp.float32(1.0 / N)
    safe = jnp.where(prob > 0, prob, jnp.float32(1.0))
    contrib = jnp.where(prob > 0, -prob * jnp.log2(safe), jnp.float32(0.0))
    o_ref[0, 0] = jnp.sum(contrib)


@jax.jit
def kernel(output_seg):
    mesh = plsc.VectorSubcoreMesh(core_axis_name="c", subcore_axis_name="s")
    sc_hist = functools.partial(
        pl.kernel,
        mesh=mesh,
        out_type=jax.ShapeDtypeStruct((NUM_WORKERS * B * QUANT,), jnp.float32),
        scratch_types=[
            pltpu.VMEM((B, CHUNK_COLS), jnp.float32),
            pltpu.VMEM((B, CHUNK_COLS), jnp.float32),
            pltpu.VMEM((B, 128), jnp.float32),
            pltpu.VMEM((HIST,), jnp.float32),
            pltpu.VMEM((B * QUANT,), jnp.float32),
            pltpu.SemaphoreType.DMA,
            pltpu.SemaphoreType.DMA,
        ],
        compiler_params=pltpu.CompilerParams(needs_layout_passes=False),
    )(_sc_hist_kernel)

    p2 = sc_hist(output_seg).reshape(NUM_WORKERS, B * QUANT)

    loss = pl.pallas_call(
        _entropy_kernel,
        out_shape=jax.ShapeDtypeStruct((1, 1), jnp.float32),
        out_specs=pl.BlockSpec(memory_space=pltpu.SMEM),
    )(p2)
    return loss[0, 0]


# speedup vs baseline: 1.1010x; 1.1010x over previous
import functools

import jax
import jax.numpy as jnp
import numpy as np
from jax import lax
from jax.experimental import pallas as pl
from jax.experimental.pallas import tpu as pltpu
from jax.experimental.pallas import tpu_sc as plsc

QUANT = 20
B = 8
N = 2_000_000
NUM_WORKERS = 32
LANE = 16

COL_TILES = N // 128
TILES_PER_W = COL_TILES // NUM_WORKERS
CT = 8
CHUNK_COLS = CT * 128
NCHUNK = TILES_PER_W // CT
NPAIR = NCHUNK // 2
HIST = B * QUANT * LANE

_INV_STEP = np.float32(1.0) / np.float32(1.0 / QUANT)


def _sc_hist_kernel(x_hbm, out_hbm, buf0, buf1, tail, hist, sem0, sem1):
    wid = lax.axis_index("s") * 2 + lax.axis_index("c")
    wbase = wid * (TILES_PER_W * 128)

    lanes = lax.iota(jnp.int32, 16)
    ones = jnp.ones((16,), jnp.float32)
    all_roffs = [lanes + _r * QUANT * 16 for _r in range(B)]

    zeros = jnp.zeros((16,), jnp.float32)

    @plsc.parallel_loop(0, HIST, step=16, unroll=4)
    def _(i):
        hist[pl.ds(i, 16)] = zeros

    def process(buf, ncols=CHUNK_COLS):
        for r0 in range(0, B, 4):
            @plsc.parallel_loop(0, ncols, step=16, unroll=8)
            def _(i):
                for k in range(r0, r0 + 4):
                    v = buf[k, pl.ds(i, 16)]
                    bins = (v * _INV_STEP).astype(jnp.int32)
                    plsc.addupdate_scatter(hist, [bins * 16 + all_roffs[k]], ones)

    def start(col, buf, sem):
        col = pl.multiple_of(col, 128)
        return pltpu.async_copy(x_hbm.at[:, pl.ds(col, CHUNK_COLS)], buf, sem)

    def drain(buf, sem):
        pltpu.make_async_copy(
            x_hbm.at[:, pl.ds(0, CHUNK_COLS)], buf, sem).wait()

    start(wbase, buf0, sem0)

    def pair_body(j, _):
        col0 = wbase + j * (2 * CHUNK_COLS)
        h1 = start(col0 + CHUNK_COLS, buf1, sem1)
        drain(buf0, sem0)
        process(buf0)
        start(col0 + 2 * CHUNK_COLS, buf0, sem0)
        h1.wait()
        process(buf1)
        return 0

    lax.fori_loop(0, NPAIR, pair_body, 0)
    drain(buf0, sem0)
    process(buf0)

    @pl.when(wid < COL_TILES - NUM_WORKERS * TILES_PER_W)
    def _():
        col = pl.multiple_of((NUM_WORKERS * TILES_PER_W + wid) * 128, 128)
        pltpu.sync_copy(x_hbm.at[:, pl.ds(col, 128)], tail)
        process(tail, 128)

    pltpu.sync_copy(hist, out_hbm.at[pl.ds(wid * HIST, HIST)])


def _entropy_kernel(p_ref, o_ref):
    s = jnp.sum(p_ref[...], axis=0)
    sel = (lax.broadcasted_iota(jnp.int32, (QUANT * LANE, QUANT), 0) // LANE
           == lax.broadcasted_iota(jnp.int32, (QUANT * LANE, QUANT), 1)
           ).astype(jnp.float32)
    counts = jax.lax.dot_general(
        s, sel, (((1,), (0,)), ((), ())),
        preferred_element_type=jnp.float32)
    prob = counts * jnp.float32(1.0 / N)
    safe = jnp.where(prob > 0, prob, jnp.float32(1.0))
    contrib = jnp.where(prob > 0, -prob * jnp.log2(safe), jnp.float32(0.0))
    o_ref[0, 0] = jnp.sum(contrib)


@jax.jit
def kernel(output_seg):
    mesh = plsc.VectorSubcoreMesh(core_axis_name="c", subcore_axis_name="s")
    sc_hist = functools.partial(
        pl.kernel,
        mesh=mesh,
        out_type=jax.ShapeDtypeStruct((NUM_WORKERS * HIST,), jnp.float32),
        scratch_types=[
            pltpu.VMEM((B, CHUNK_COLS), jnp.float32),
            pltpu.VMEM((B, CHUNK_COLS), jnp.float32),
            pltpu.VMEM((B, 128), jnp.float32),
            pltpu.VMEM((HIST,), jnp.float32),
            pltpu.SemaphoreType.DMA,
            pltpu.SemaphoreType.DMA,
        ],
        compiler_params=pltpu.CompilerParams(needs_layout_passes=False),
    )(_sc_hist_kernel)

    p2 = sc_hist(output_seg).reshape(NUM_WORKERS, B, QUANT * LANE)

    loss = pl.pallas_call(
        _entropy_kernel,
        out_shape=jax.ShapeDtypeStruct((1, 1), jnp.float32),
        out_specs=pl.BlockSpec(memory_space=pltpu.SMEM),
    )(p2)
    return loss[0, 0]
